# baseline (speedup 1.0000x reference)
import functools

import jax
import jax.numpy as jnp
from jax import lax
from jax.experimental import pallas as pl
from jax.experimental.pallas import tpu as pltpu
from jax.experimental.pallas import tpu_sc as plsc

_B, _C = 16384, 1000
_NW = 32
_CPW = _B // _NW
_CHR = 40
_NCH = _C // _CHR
_NBUF = 4
_PF = 2


def kernel(input, masked_values):
    inp_t = input.T
    mesh = plsc.VectorSubcoreMesh(core_axis_name="c", subcore_axis_name="s")

    @functools.partial(
        pl.kernel,
        mesh=mesh,
        compiler_params=pltpu.CompilerParams(
            use_tc_tiling_on_sc=True, needs_layout_passes=False
        ),
        out_type=jax.ShapeDtypeStruct((_C, _B), jnp.float32),
        scratch_types=[
            pltpu.VMEM((_CPW,), jnp.int32),
            [pltpu.VMEM((_CHR, _CPW), jnp.float32) for _ in range(_NBUF)],
            [pltpu.SemaphoreType.DMA for _ in range(_NBUF)],
            [pltpu.SemaphoreType.DMA for _ in range(_NBUF)],
        ],
    )
    def sc(in_hbm, mv_hbm, out_hbm, mv_v, bufs, in_sems, out_sems):
        wid = lax.axis_index("s") * 2 + lax.axis_index("c")
        base = wid * _CPW
        pltpu.sync_copy(mv_hbm.at[pl.ds(base, _CPW)], mv_v)
        neg1 = jnp.full((16,), -1.0, jnp.float32)

        cin = [None] * _NBUF
        cout = [None] * _NBUF

        def start_in(g):
            s = g % _NBUF
            cin[s] = pltpu.async_copy(
                in_hbm.at[pl.ds(g * _CHR, _CHR), pl.ds(base, _CPW)],
                bufs[s],
                in_sems[s],
            )

        for j in range(min(_PF, _NCH)):
            start_in(j)
        for g in range(_NCH):
            s = g % _NBUF
            pf = g + _PF
            if pf < _NCH:
                if pf >= _NBUF:
                    cout[pf % _NBUF].wait()
                start_in(pf)
            cin[s].wait()
            j0 = g * _CHR
            for k in range(_CPW // 16):
                cols = lax.iota(jnp.int32, 16) + (k * 16)
                mvv = mv_v[pl.ds(k * 16, 16)]
                rows = mvv - j0
                hit = (mvv >= j0) & (mvv < j0 + _CHR)
                plsc.store_scatter(bufs[s], [rows, cols], neg1, mask=hit)
            cout[s] = pltpu.async_copy(
                bufs[s],
                out_hbm.at[pl.ds(j0, _CHR), pl.ds(base, _CPW)],
                out_sems[s],
            )
        for g in range(_NCH - _NBUF, _NCH):
            cout[g % _NBUF].wait()

    return sc(inp_t, masked_values).T


# speedup vs baseline: 1.0201x; 1.0201x over previous
import functools

import jax
import jax.numpy as jnp
from jax import lax
from jax.experimental import pallas as pl
from jax.experimental.pallas import tpu as pltpu
from jax.experimental.pallas import tpu_sc as plsc

_B, _C = 16384, 1000
_NW = 32
_CPW = _B // _NW
_CHR = 40
_NCH = _C // _CHR
_NBUF = 6
_PF = 4


def kernel(input, masked_values):
    inp_t = input.T
    mesh = plsc.VectorSubcoreMesh(core_axis_name="c", subcore_axis_name="s")

    @functools.partial(
        pl.kernel,
        mesh=mesh,
        compiler_params=pltpu.CompilerParams(
            use_tc_tiling_on_sc=True, needs_layout_passes=False
        ),
        out_type=jax.ShapeDtypeStruct((_C, _B), jnp.float32),
        scratch_types=[
            pltpu.VMEM((_CPW,), jnp.int32),
            [pltpu.VMEM((_CHR, _CPW), jnp.float32) for _ in range(_NBUF)],
            [pltpu.SemaphoreType.DMA for _ in range(_NBUF)],
            [pltpu.SemaphoreType.DMA for _ in range(_NBUF)],
        ],
    )
    def sc(in_hbm, mv_hbm, out_hbm, mv_v, bufs, in_sems, out_sems):
        wid = lax.axis_index("s") * 2 + lax.axis_index("c")
        base = wid * _CPW
        pltpu.sync_copy(mv_hbm.at[pl.ds(base, _CPW)], mv_v)
        neg1 = jnp.full((16,), -1.0, jnp.float32)

        cin = [None] * _NBUF
        cout = [None] * _NBUF

        def start_in(g):
            s = g % _NBUF
            cin[s] = pltpu.async_copy(
                in_hbm.at[pl.ds(g * _CHR, _CHR), pl.ds(base, _CPW)],
                bufs[s],
                in_sems[s],
            )

        for j in range(min(_PF, _NCH)):
            start_in(j)
        for g in range(_NCH):
            s = g % _NBUF
            pf = g + _PF
            if pf < _NCH:
                if pf >= _NBUF:
                    cout[pf % _NBUF].wait()
                start_in(pf)
            cin[s].wait()
            j0 = g * _CHR
            for k in range(_CPW // 16):
                cols = lax.iota(jnp.int32, 16) + (k * 16)
                mvv = mv_v[pl.ds(k * 16, 16)]
                rows = mvv - j0
                hit = (mvv >= j0) & (mvv < j0 + _CHR)
                plsc.store_scatter(bufs[s], [rows, cols], neg1, mask=hit)
            cout[s] = pltpu.async_copy(
                bufs[s],
                out_hbm.at[pl.ds(j0, _CHR), pl.ds(base, _CPW)],
                out_sems[s],
            )
        for g in range(_NCH - _NBUF, _NCH):
            cout[g % _NBUF].wait()

    return sc(inp_t, masked_values).T


# speedup vs baseline: 1.8623x; 1.8255x over previous
import jax
import jax.numpy as jnp
from jax.experimental import pallas as pl
from jax.experimental.pallas import tpu as pltpu

_BC = 4096


def _mask_body(x_ref, mv_ref, o_ref):
    x = x_ref[...]
    mv = mv_ref[0, 0, :]
    row = jax.lax.broadcasted_iota(jnp.int32, x.shape, 0)
    o_ref[...] = jnp.where(row == mv[None, :], jnp.float32(-1.0), x)


def kernel(input, masked_values):
    B, C = input.shape
    inp_t = input.T
    grid = (B // _BC,)
    mv3 = masked_values.reshape(grid[0], 1, _BC)
    out_t = pl.pallas_call(
        _mask_body,
        grid=grid,
        compiler_params=pltpu.CompilerParams(vmem_limit_bytes=100 * 1024 * 1024),
        in_specs=[
            pl.BlockSpec((C, _BC), lambda i: (0, i)),
            pl.BlockSpec((1, 1, _BC), lambda i: (i, 0, 0)),
        ],
        out_specs=pl.BlockSpec((C, _BC), lambda i: (0, i)),
        out_shape=jax.ShapeDtypeStruct((C, B), input.dtype),
    )(inp_t, mv3)
    return out_t.T
